# baseline (speedup 1.0000x reference)
import functools

import numpy as np

import jax
import jax.numpy as jnp
from jax import lax
from jax.experimental import pallas as pl
from jax.experimental.pallas import tpu as pltpu
from jax.experimental.pallas import tpu_sc as plsc

NC = 2
NS = 16
NW = NC * NS
CHUNK = 80


def _make_edge_agg(N, E, D, loff):
    n_chunks = E // CHUNK
    per_w = n_chunks // NW
    assert per_w * NW == n_chunks
    rows_per_tile = (N // (8 * NS)) * 8
    rem = N - NS * rows_per_tile
    assert rem % 8 == 0 and rem <= CHUNK
    pieces = []
    off = 0
    while off < rows_per_tile:
        pc = min(CHUNK, rows_per_tile - off)
        assert pc % 8 == 0
        pieces.append((off, pc))
        off += pc

    mesh = plsc.VectorSubcoreMesh(core_axis_name="c", subcore_axis_name="s")

    @functools.partial(
        pl.kernel,
        out_type=jax.ShapeDtypeStruct((2 * N, D), jnp.float32),
        mesh=mesh,
        scratch_types=[
            pltpu.VMEM((CHUNK,), jnp.int32),
            pltpu.VMEM((CHUNK,), jnp.int32),
            pltpu.VMEM((CHUNK,), jnp.int32),
            pltpu.VMEM((CHUNK,), jnp.int32),
            pltpu.VMEM((CHUNK, D), jnp.float32),
            pltpu.VMEM((CHUNK, D), jnp.float32),
            pltpu.VMEM((CHUNK // 2, D), jnp.int32),
            pltpu.VMEM((CHUNK // 2, D), jnp.int32),
            pltpu.VMEM_SHARED((N, D), jnp.float32),
            pltpu.SemaphoreType.DMA,
            pltpu.SemaphoreType.DMA,
            pltpu.SemaphoreType.DMA,
            pltpu.SemaphoreType.DMA,
            pltpu.SemaphoreType.DMA,
            pltpu.SemaphoreType.DMA,
            pltpu.SemaphoreType.DMA,
            pltpu.SemaphoreType.DMA,
        ],
    )
    def edge_agg(h_hbm, ee_hbm, src_hbm, dst_hbm, out_hbm,
                 src0, src1, dst0, dst1, grow0, grow1, ee0, ee1, acc_sh,
                 ss0, ss1, sd0, sd1, sg0, sg1, se0, se1):
        c = lax.axis_index("c")
        s = lax.axis_index("s")
        w = c * NS + s
        srcs, dsts, grows, ees = (src0, src1), (dst0, dst1), \
            (grow0, grow1), (ee0, ee1)
        sss, sds, sgs, ses = (ss0, ss1), (sd0, sd1), (sg0, sg1), (se0, se1)

        zv = jnp.zeros((16,), jnp.float32)

        def zrow(i, carry):
            for u in range(D // 16):
                grow0[i, pl.ds(u * 16, 16)] = zv
            return carry

        lax.fori_loop(0, CHUNK, zrow, 0)
        base = s * rows_per_tile
        for po, pc in pieces:
            pltpu.sync_copy(grow0.at[pl.ds(0, pc)],
                            acc_sh.at[pl.ds(base + po, pc)])
        if rem:
            @pl.when(s == NS - 1)
            def _():
                pltpu.sync_copy(grow0.at[pl.ds(0, rem)],
                                acc_sh.at[pl.ds(NS * rows_per_tile, rem)])
        plsc.subcore_barrier()

        def ebase(r):
            return (w * per_w + r) * CHUNK

        def eebase(r):
            return loff // 2 + (w * per_w + r) * (CHUNK // 2)

        def idx_issue(r, sl):
            pltpu.async_copy(src_hbm.at[pl.ds(ebase(r), CHUNK)],
                             srcs[sl], sss[sl])
            pltpu.async_copy(dst_hbm.at[pl.ds(ebase(r), CHUNK)],
                             dsts[sl], sds[sl])

        def idx_wait(sl):
            pltpu.make_async_copy(src_hbm.at[pl.ds(0, CHUNK)],
                                  srcs[sl], sss[sl]).wait()
            pltpu.make_async_copy(dst_hbm.at[pl.ds(0, CHUNK)],
                                  dsts[sl], sds[sl]).wait()

        def ge_issue(r, sl):
            pltpu.async_copy(ee_hbm.at[pl.ds(eebase(r), CHUNK // 2)],
                             ees[sl], ses[sl])
            pltpu.async_copy(h_hbm.at[srcs[sl]], grows[sl], sgs[sl])

        def ge_wait(sl):
            pltpu.make_async_copy(ee_hbm.at[pl.ds(0, CHUNK // 2)],
                                  ees[sl], ses[sl]).wait()
            pltpu.make_async_copy(h_hbm.at[srcs[sl]],
                                  grows[sl], sgs[sl]).wait()

        def step(r, cur):
            nxt = 1 - cur

            @pl.when(r < per_w - 1)
            def _():
                idx_wait(nxt)
                ge_issue(r + 1, nxt)

            ge_wait(cur)
            gv, ev = grows[cur], ees[cur]
            himask = jnp.int32(-65536)

            def crow(q, carry):
                i = q * 2
                for u in range(D // 16):
                    sl = pl.ds(u * 16, 16)
                    wv = ev[q, sl]
                    lo = lax.bitcast_convert_type(
                        lax.shift_left(wv, 16), jnp.float32)
                    hi = lax.bitcast_convert_type(wv & himask, jnp.float32)
                    gv[i, sl] = jnp.maximum(gv[i, sl] + lo, 0.0)
                    gv[i + 1, sl] = jnp.maximum(gv[i + 1, sl] + hi, 0.0)
                return carry

            lax.fori_loop(0, CHUNK // 2, crow, 0)
            pltpu.sync_copy(gv, acc_sh.at[dsts[cur]], add=True)

            @pl.when(r < per_w - 2)
            def _():
                idx_issue(r + 2, cur)

        idx_issue(0, 0)
        idx_wait(0)
        ge_issue(0, 0)
        idx_issue(1, 1)

        def body(r, carry):
            @pl.when(lax.rem(r, 2) == 0)
            def _():
                step(r, 0)

            @pl.when(lax.rem(r, 2) == 1)
            def _():
                step(r, 1)

            return carry

        lax.fori_loop(0, per_w, body, 0)

        plsc.subcore_barrier()
        for po, pc in pieces:
            pltpu.sync_copy(acc_sh.at[pl.ds(base + po, pc)],
                            out_hbm.at[pl.ds(c * N + base + po, pc)])
        if rem:
            @pl.when(s == NS - 1)
            def _():
                pltpu.sync_copy(acc_sh.at[pl.ds(NS * rows_per_tile, rem)],
                                out_hbm.at[pl.ds(c * N + NS * rows_per_tile, rem)])

    return edge_agg


def _make_gather_rows(N, G, D):
    mesh = plsc.VectorSubcoreMesh(core_axis_name="c", subcore_axis_name="s")

    @functools.partial(
        pl.kernel,
        out_type=jax.ShapeDtypeStruct((G, D), jnp.float32),
        mesh=mesh,
        scratch_types=[
            pltpu.VMEM((G,), jnp.int32),
            pltpu.VMEM((G, D), jnp.float32),
            pltpu.SemaphoreType.DMA,
        ],
    )
    def gather_rows(tab_hbm, idx_hbm, out_hbm, idx_v, rows_v, sem):
        c = lax.axis_index("c")
        s = lax.axis_index("s")

        @pl.when(jnp.logical_and(c == 0, s == 0))
        def _():
            pltpu.sync_copy(idx_hbm, idx_v)
            pltpu.async_copy(tab_hbm.at[idx_v], rows_v, sem).wait()
            pltpu.sync_copy(rows_v, out_hbm)

    return gather_rows


def _ee_matmul(ea_even_t, ea_odd_t, We):
    L, DE, D = We.shape
    Eh = ea_even_t.shape[1]
    BE = 1280
    assert Eh % BE == 0
    nj = Eh // BE

    def bf16_bits(y):
        i = lax.bitcast_convert_type(y, jnp.int32)
        return lax.shift_right_logical(
            i + jnp.int32(0x7FFF) + (lax.shift_right_logical(i, 16) & 1), 16)

    def body(eae_ref, eao_ref, we_ref, out_ref):
        dn = (((0,), (0,)), ((), ()))
        ye = lax.dot_general(eae_ref[...], we_ref[0], dn,
                             preferred_element_type=jnp.float32)
        yo = lax.dot_general(eao_ref[...], we_ref[0], dn,
                             preferred_element_type=jnp.float32)
        out_ref[...] = bf16_bits(ye) | lax.shift_left(bf16_bits(yo), 16)

    return pl.pallas_call(
        body,
        grid=(nj, L),
        in_specs=[
            pl.BlockSpec((DE, BE), lambda j, k: (0, j)),
            pl.BlockSpec((DE, BE), lambda j, k: (0, j)),
            pl.BlockSpec((1, DE, D), lambda j, k: (k, 0, 0)),
        ],
        out_specs=pl.BlockSpec((BE, D), lambda j, k: (k * nj + j, 0)),
        out_shape=jax.ShapeDtypeStruct((L * Eh, D), jnp.int32),
    )(ea_even_t, ea_odd_t, We)


def _node_update(h, parts, wl, bl, N, D, BN, wl_p=None, bl_p=None):
    nb = N // BN
    emit_bf = wl_p is not None

    def body(h_ref, a_ref, b_ref, w_ref, b2_ref, *rest):
        acc = h_ref[...] + a_ref[...] + b_ref[...]
        if emit_bf:
            wp_ref, bp_ref, out_ref, outb_ref = rest
            outb_ref[...] = jnp.maximum(
                jnp.dot(acc, wp_ref[...], preferred_element_type=jnp.float32)
                + bp_ref[...], 0.0).astype(jnp.bfloat16)
        else:
            (out_ref,) = rest
        out_ref[...] = jnp.maximum(
            jnp.dot(acc, w_ref[...], preferred_element_type=jnp.float32)
            + b2_ref[...], 0.0)

    in_specs = [
        pl.BlockSpec((BN, D), lambda j: (j, 0)),
        pl.BlockSpec((BN, D), lambda j: (j, 0)),
        pl.BlockSpec((BN, D), lambda j: (nb + j, 0)),
        pl.BlockSpec((D, D), lambda j: (0, 0)),
        pl.BlockSpec((1, D), lambda j: (0, 0)),
    ]
    args = [h, parts, parts, wl, bl]
    out_specs = pl.BlockSpec((BN, D), lambda j: (j, 0))
    out_shape = jax.ShapeDtypeStruct((N, D), jnp.float32)
    if emit_bf:
        in_specs += [pl.BlockSpec((D, D), lambda j: (0, 0)),
                     pl.BlockSpec((1, D), lambda j: (0, 0))]
        args += [wl_p, bl_p]
        out_specs = [out_specs, pl.BlockSpec((BN, D), lambda j: (j, 0))]
        out_shape = [out_shape, jax.ShapeDtypeStruct((N, D), jnp.bfloat16)]

    return pl.pallas_call(
        body, grid=(nb,), in_specs=in_specs,
        out_specs=out_specs, out_shape=out_shape,
    )(*args)


def _node_update_pool(h, parts, wl, bl, batch2, N, D, G, BN):
    nb = N // BN

    def body(h_ref, a_ref, b_ref, w_ref, b2_ref, bt_ref,
             out_ref, gsum_ref, cnt_ref):
        acc = h_ref[...] + a_ref[...] + b_ref[...]
        x3 = jnp.maximum(
            jnp.dot(acc, w_ref[...], preferred_element_type=jnp.float32)
            + b2_ref[...], 0.0)
        out_ref[...] = x3
        gi = lax.broadcasted_iota(jnp.int32, (BN, G), 1)
        oh = (bt_ref[...] == gi).astype(jnp.float32)

        @pl.when(pl.program_id(0) == 0)
        def _():
            gsum_ref[...] = jnp.zeros((G, D), jnp.float32)
            cnt_ref[...] = jnp.zeros((G, D), jnp.float32)

        gsum_ref[...] += lax.dot_general(
            oh, x3, (((0,), (0,)), ((), ())),
            preferred_element_type=jnp.float32)
        cnt_ref[...] += jnp.broadcast_to(
            jnp.sum(oh, axis=0)[:, None], (G, D))

    return pl.pallas_call(
        body,
        grid=(nb,),
        in_specs=[
            pl.BlockSpec((BN, D), lambda j: (j, 0)),
            pl.BlockSpec((BN, D), lambda j: (j, 0)),
            pl.BlockSpec((BN, D), lambda j: (nb + j, 0)),
            pl.BlockSpec((D, D), lambda j: (0, 0)),
            pl.BlockSpec((1, D), lambda j: (0, 0)),
            pl.BlockSpec((BN, 1), lambda j: (j, 0)),
        ],
        out_specs=[
            pl.BlockSpec((BN, D), lambda j: (j, 0)),
            pl.BlockSpec((G, D), lambda j: (0, 0)),
            pl.BlockSpec((G, D), lambda j: (0, 0)),
        ],
        out_shape=[
            jax.ShapeDtypeStruct((N, D), jnp.float32),
            jax.ShapeDtypeStruct((G, D), jnp.float32),
            jax.ShapeDtypeStruct((G, D), jnp.float32),
        ],
    )(h, parts, parts, wl, bl, batch2)


def _reset(x_orig, batch2, xc, wr_top, wr_bot, br, N, D, G, BN):
    nb = N // BN

    def body(xo_ref, bt_ref, xc_ref, wt_ref, wb_ref, br_ref, out_ref):
        gi = lax.broadcasted_iota(jnp.int32, (BN, G), 1)
        oh = (bt_ref[...] == gi).astype(jnp.float32)
        y2 = jnp.dot(xc_ref[...], wb_ref[...],
                     preferred_element_type=jnp.float32)
        cond = jnp.dot(oh, y2, preferred_element_type=jnp.float32)
        out_ref[...] = jnp.maximum(
            jnp.dot(xo_ref[...], wt_ref[...],
                    preferred_element_type=jnp.float32)
            + cond + br_ref[...], 0.0)

    full = lambda j: (0, 0)
    return pl.pallas_call(
        body,
        grid=(nb,),
        in_specs=[
            pl.BlockSpec((BN, D), lambda j: (j, 0)),
            pl.BlockSpec((BN, 1), lambda j: (j, 0)),
            pl.BlockSpec((G, D), full),
            pl.BlockSpec((D, D), full),
            pl.BlockSpec((D, D), full),
            pl.BlockSpec((1, D), full),
        ],
        out_specs=pl.BlockSpec((BN, D), lambda j: (j, 0)),
        out_shape=jax.ShapeDtypeStruct((N, D), jnp.float32),
    )(x_orig, batch2, xc, wr_top, wr_bot, br)


def _center_update(xc3, gsum, cnt, Wc, bc, Wm, bm, G, D):
    def body(xc_ref, gs_ref, ct_ref, wc_ref, bc_ref, wm_ref, bm_ref, out_ref):
        gmp = gs_ref[...] / jnp.maximum(ct_ref[...], 1.0)
        out_ref[...] = jnp.maximum(
            jnp.dot(xc_ref[...], wc_ref[...], preferred_element_type=jnp.float32)
            + bc_ref[...]
            + jnp.dot(gmp, wm_ref[...], preferred_element_type=jnp.float32)
            + bm_ref[...], 0.0)

    return pl.pallas_call(
        body,
        out_shape=jax.ShapeDtypeStruct((G, D), jnp.float32),
    )(xc3, gsum, cnt, Wc, bc, Wm, bm)


def _overwrite(x3, cni2, center, N, D, G, BN):
    nb = N // BN

    def body(x3_ref, cni_ref, c_ref, out_ref):
        j = pl.program_id(0)
        rows = j * BN + lax.broadcasted_iota(jnp.int32, (BN, G), 0)
        m = rows == cni_ref[...]
        gi = lax.broadcasted_iota(jnp.int32, (BN, G), 1)
        gsel = jnp.max(jnp.where(m, gi, -1), axis=1)
        has = gsel >= 0
        oh2 = (gi == gsel[:, None]).astype(jnp.float32) * \
            has[:, None].astype(jnp.float32)
        repl = jnp.dot(oh2, c_ref[...], preferred_element_type=jnp.float32)
        out_ref[...] = jnp.where(has[:, None], repl, x3_ref[...])

    return pl.pallas_call(
        body,
        grid=(nb,),
        in_specs=[
            pl.BlockSpec((BN, D), lambda j: (j, 0)),
            pl.BlockSpec((1, G), lambda j: (0, 0)),
            pl.BlockSpec((G, D), lambda j: (0, 0)),
        ],
        out_specs=pl.BlockSpec((BN, D), lambda j: (j, 0)),
        out_shape=jax.ShapeDtypeStruct((N, D), jnp.float32),
    )(x3, cni2, center)


def kernel(x_orig, x, edge_index, edge_attr, center_node_index, batch,
           We, Wl, bl, Wr, br, Wc, bc, Wm, bm):
    N, D = x.shape
    E = edge_attr.shape[0]
    G = center_node_index.shape[0]
    L = We.shape[0]
    BN = 2000
    assert N % BN == 0 and E % CHUNK == 0

    src = edge_index[0]
    dst = edge_index[1]
    batch2 = batch.reshape(N, 1)
    cni2 = center_node_index.reshape(1, G)

    gather_rows = _make_gather_rows(N, G, D)
    aggs = [_make_edge_agg(N, E, D, k * E) for k in range(L)]

    ee_all = _ee_matmul(edge_attr[0::2].T, edge_attr[1::2].T, We)

    h = x
    parts = aggs[0](h, ee_all, src, dst)
    x1 = _node_update(h, parts, Wl[0], bl[0].reshape(1, D), N, D, BN)
    xc1 = gather_rows(x1, center_node_index)
    xr = _reset(x_orig, batch2, xc1, Wr[:D], Wr[D:], br.reshape(1, D),
                N, D, G, BN)
    parts = aggs[1](xr, ee_all, src, dst)
    x2 = _node_update(xr, parts, Wl[1], bl[1].reshape(1, D), N, D, BN)
    parts = aggs[2](x2, ee_all, src, dst)
    x3, gsum, cnt = _node_update_pool(x2, parts, Wl[2], bl[2].reshape(1, D),
                                      batch2, N, D, G, BN)
    xc3 = gather_rows(x3, center_node_index)
    center = _center_update(xc3, gsum, cnt, Wc, bc.reshape(1, D),
                            Wm, bm.reshape(1, D), G, D)
    return _overwrite(x3, cni2, center, N, D, G, BN)


# speedup vs baseline: 2.2001x; 2.2001x over previous
import functools

import numpy as np

import jax
import jax.numpy as jnp
from jax import lax
from jax.experimental import pallas as pl
from jax.experimental.pallas import tpu as pltpu
from jax.experimental.pallas import tpu_sc as plsc

NC = 2
NS = 16
NW = NC * NS
CHUNK = 80


def _make_edge_agg(N, E, D):
    n_chunks = E // CHUNK
    per_w = n_chunks // NW
    assert per_w * NW == n_chunks
    rows_per_tile = (N // (8 * NS)) * 8
    rem = N - NS * rows_per_tile
    assert rem % 8 == 0 and rem <= CHUNK
    pieces = []
    off = 0
    while off < rows_per_tile:
        pc = min(CHUNK, rows_per_tile - off)
        assert pc % 8 == 0
        pieces.append((off, pc))
        off += pc

    mesh = plsc.VectorSubcoreMesh(core_axis_name="c", subcore_axis_name="s")

    @functools.partial(
        pl.kernel,
        out_type=jax.ShapeDtypeStruct((2 * N, D), jnp.float32),
        mesh=mesh,
        scratch_types=[
            pltpu.VMEM((CHUNK,), jnp.int32),
            pltpu.VMEM((CHUNK,), jnp.int32),
            pltpu.VMEM((CHUNK,), jnp.int32),
            pltpu.VMEM((CHUNK,), jnp.int32),
            pltpu.VMEM((CHUNK, D), jnp.float32),
            pltpu.VMEM((CHUNK, D), jnp.float32),
            pltpu.VMEM((CHUNK, D), jnp.float32),
            pltpu.VMEM((CHUNK, D), jnp.float32),
            pltpu.VMEM_SHARED((N, D), jnp.float32),
            pltpu.SemaphoreType.DMA,
            pltpu.SemaphoreType.DMA,
            pltpu.SemaphoreType.DMA,
            pltpu.SemaphoreType.DMA,
            pltpu.SemaphoreType.DMA,
            pltpu.SemaphoreType.DMA,
            pltpu.SemaphoreType.DMA,
            pltpu.SemaphoreType.DMA,
        ],
    )
    def edge_agg(h_hbm, ee_hbm, src_hbm, dst_hbm, out_hbm,
                 src0, src1, dst0, dst1, grow0, grow1, ee0, ee1, acc_sh,
                 ss0, ss1, sd0, sd1, sg0, sg1, se0, se1):
        c = lax.axis_index("c")
        s = lax.axis_index("s")
        w = c * NS + s
        srcs, dsts, grows, ees = (src0, src1), (dst0, dst1), \
            (grow0, grow1), (ee0, ee1)
        sss, sds, sgs, ses = (ss0, ss1), (sd0, sd1), (sg0, sg1), (se0, se1)

        zv = jnp.zeros((16,), jnp.float32)

        def zrow(i, carry):
            for u in range(D // 16):
                grow0[i, pl.ds(u * 16, 16)] = zv
            return carry

        lax.fori_loop(0, CHUNK, zrow, 0)
        base = s * rows_per_tile
        for po, pc in pieces:
            pltpu.sync_copy(grow0.at[pl.ds(0, pc)],
                            acc_sh.at[pl.ds(base + po, pc)])
        if rem:
            @pl.when(s == NS - 1)
            def _():
                pltpu.sync_copy(grow0.at[pl.ds(0, rem)],
                                acc_sh.at[pl.ds(NS * rows_per_tile, rem)])
        plsc.subcore_barrier()

        def ebase(r):
            return (w * per_w + r) * CHUNK

        def idx_issue(r, sl):
            pltpu.async_copy(src_hbm.at[pl.ds(ebase(r), CHUNK)],
                             srcs[sl], sss[sl])
            pltpu.async_copy(dst_hbm.at[pl.ds(ebase(r), CHUNK)],
                             dsts[sl], sds[sl])

        def idx_wait(sl):
            pltpu.make_async_copy(src_hbm.at[pl.ds(0, CHUNK)],
                                  srcs[sl], sss[sl]).wait()
            pltpu.make_async_copy(dst_hbm.at[pl.ds(0, CHUNK)],
                                  dsts[sl], sds[sl]).wait()

        def ge_issue(r, sl):
            pltpu.async_copy(ee_hbm.at[pl.ds(ebase(r), CHUNK)],
                             ees[sl], ses[sl])
            pltpu.async_copy(h_hbm.at[srcs[sl]], grows[sl], sgs[sl])

        def ge_wait(sl):
            pltpu.make_async_copy(ee_hbm.at[pl.ds(0, CHUNK)],
                                  ees[sl], ses[sl]).wait()
            pltpu.make_async_copy(h_hbm.at[srcs[sl]],
                                  grows[sl], sgs[sl]).wait()

        def step(r, cur):
            nxt = 1 - cur

            @pl.when(r < per_w - 1)
            def _():
                idx_wait(nxt)
                ge_issue(r + 1, nxt)

            ge_wait(cur)
            gv, ev = grows[cur], ees[cur]

            def crow(i, carry):
                for u in range(D // 16):
                    sl = pl.ds(u * 16, 16)
                    gv[i, sl] = jnp.maximum(gv[i, sl] + ev[i, sl], 0.0)
                return carry

            lax.fori_loop(0, CHUNK, crow, 0)
            pltpu.sync_copy(gv, acc_sh.at[dsts[cur]], add=True)

            @pl.when(r < per_w - 2)
            def _():
                idx_issue(r + 2, cur)

        idx_issue(0, 0)
        idx_wait(0)
        ge_issue(0, 0)
        idx_issue(1, 1)

        def body(r, carry):
            @pl.when(lax.rem(r, 2) == 0)
            def _():
                step(r, 0)

            @pl.when(lax.rem(r, 2) == 1)
            def _():
                step(r, 1)

            return carry

        lax.fori_loop(0, per_w, body, 0)

        plsc.subcore_barrier()
        for po, pc in pieces:
            pltpu.sync_copy(acc_sh.at[pl.ds(base + po, pc)],
                            out_hbm.at[pl.ds(c * N + base + po, pc)])
        if rem:
            @pl.when(s == NS - 1)
            def _():
                pltpu.sync_copy(acc_sh.at[pl.ds(NS * rows_per_tile, rem)],
                                out_hbm.at[pl.ds(c * N + NS * rows_per_tile, rem)])

    return edge_agg


def _make_gather_rows(N, G, D):
    mesh = plsc.VectorSubcoreMesh(core_axis_name="c", subcore_axis_name="s")

    @functools.partial(
        pl.kernel,
        out_type=jax.ShapeDtypeStruct((G, D), jnp.float32),
        mesh=mesh,
        scratch_types=[
            pltpu.VMEM((G,), jnp.int32),
            pltpu.VMEM((G, D), jnp.float32),
            pltpu.SemaphoreType.DMA,
        ],
    )
    def gather_rows(tab_hbm, idx_hbm, out_hbm, idx_v, rows_v, sem):
        c = lax.axis_index("c")
        s = lax.axis_index("s")

        @pl.when(jnp.logical_and(c == 0, s == 0))
        def _():
            pltpu.sync_copy(idx_hbm, idx_v)
            pltpu.async_copy(tab_hbm.at[idx_v], rows_v, sem).wait()
            pltpu.sync_copy(rows_v, out_hbm)

    return gather_rows


def _ee_matmul(edge_attr_t, we_k):
    DE, E = edge_attr_t.shape
    D = we_k.shape[1]
    BE = 2560
    assert E % BE == 0
    nj = E // BE

    def body(ea_ref, we_ref, out_ref):
        out_ref[...] = lax.dot_general(
            ea_ref[...], we_ref[...], (((0,), (0,)), ((), ())),
            preferred_element_type=jnp.float32)

    return pl.pallas_call(
        body,
        grid=(nj,),
        in_specs=[
            pl.BlockSpec((DE, BE), lambda j: (0, j)),
            pl.BlockSpec((DE, D), lambda j: (0, 0)),
        ],
        out_specs=pl.BlockSpec((BE, D), lambda j: (j, 0)),
        out_shape=jax.ShapeDtypeStruct((E, D), jnp.float32),
    )(edge_attr_t, we_k)


def _node_update(h, parts, wl, bl, N, D, BN, wl_p=None, bl_p=None):
    nb = N // BN
    emit_bf = wl_p is not None

    def body(h_ref, a_ref, b_ref, w_ref, b2_ref, *rest):
        acc = h_ref[...] + a_ref[...] + b_ref[...]
        if emit_bf:
            wp_ref, bp_ref, out_ref, outb_ref = rest
            outb_ref[...] = jnp.maximum(
                jnp.dot(acc, wp_ref[...], preferred_element_type=jnp.float32)
                + bp_ref[...], 0.0).astype(jnp.bfloat16)
        else:
            (out_ref,) = rest
        out_ref[...] = jnp.maximum(
            jnp.dot(acc, w_ref[...], preferred_element_type=jnp.float32)
            + b2_ref[...], 0.0)

    in_specs = [
        pl.BlockSpec((BN, D), lambda j: (j, 0)),
        pl.BlockSpec((BN, D), lambda j: (j, 0)),
        pl.BlockSpec((BN, D), lambda j: (nb + j, 0)),
        pl.BlockSpec((D, D), lambda j: (0, 0)),
        pl.BlockSpec((1, D), lambda j: (0, 0)),
    ]
    args = [h, parts, parts, wl, bl]
    out_specs = pl.BlockSpec((BN, D), lambda j: (j, 0))
    out_shape = jax.ShapeDtypeStruct((N, D), jnp.float32)
    if emit_bf:
        in_specs += [pl.BlockSpec((D, D), lambda j: (0, 0)),
                     pl.BlockSpec((1, D), lambda j: (0, 0))]
        args += [wl_p, bl_p]
        out_specs = [out_specs, pl.BlockSpec((BN, D), lambda j: (j, 0))]
        out_shape = [out_shape, jax.ShapeDtypeStruct((N, D), jnp.bfloat16)]

    return pl.pallas_call(
        body, grid=(nb,), in_specs=in_specs,
        out_specs=out_specs, out_shape=out_shape,
    )(*args)


def _node_update_pool(h, parts, wl, bl, batch2, N, D, G, BN):
    nb = N // BN

    def body(h_ref, a_ref, b_ref, w_ref, b2_ref, bt_ref,
             out_ref, gsum_ref, cnt_ref):
        acc = h_ref[...] + a_ref[...] + b_ref[...]
        x3 = jnp.maximum(
            jnp.dot(acc, w_ref[...], preferred_element_type=jnp.float32)
            + b2_ref[...], 0.0)
        out_ref[...] = x3
        gi = lax.broadcasted_iota(jnp.int32, (BN, G), 1)
        oh = (bt_ref[...] == gi).astype(jnp.float32)

        @pl.when(pl.program_id(0) == 0)
        def _():
            gsum_ref[...] = jnp.zeros((G, D), jnp.float32)
            cnt_ref[...] = jnp.zeros((G, D), jnp.float32)

        gsum_ref[...] += lax.dot_general(
            oh, x3, (((0,), (0,)), ((), ())),
            preferred_element_type=jnp.float32)
        cnt_ref[...] += jnp.broadcast_to(
            jnp.sum(oh, axis=0)[:, None], (G, D))

    return pl.pallas_call(
        body,
        grid=(nb,),
        in_specs=[
            pl.BlockSpec((BN, D), lambda j: (j, 0)),
            pl.BlockSpec((BN, D), lambda j: (j, 0)),
            pl.BlockSpec((BN, D), lambda j: (nb + j, 0)),
            pl.BlockSpec((D, D), lambda j: (0, 0)),
            pl.BlockSpec((1, D), lambda j: (0, 0)),
            pl.BlockSpec((BN, 1), lambda j: (j, 0)),
        ],
        out_specs=[
            pl.BlockSpec((BN, D), lambda j: (j, 0)),
            pl.BlockSpec((G, D), lambda j: (0, 0)),
            pl.BlockSpec((G, D), lambda j: (0, 0)),
        ],
        out_shape=[
            jax.ShapeDtypeStruct((N, D), jnp.float32),
            jax.ShapeDtypeStruct((G, D), jnp.float32),
            jax.ShapeDtypeStruct((G, D), jnp.float32),
        ],
    )(h, parts, parts, wl, bl, batch2)


def _reset(x_orig, batch2, xc, wr_top, wr_bot, br, N, D, G, BN):
    nb = N // BN

    def body(xo_ref, bt_ref, xc_ref, wt_ref, wb_ref, br_ref, out_ref):
        gi = lax.broadcasted_iota(jnp.int32, (BN, G), 1)
        oh = (bt_ref[...] == gi).astype(jnp.float32)
        y2 = jnp.dot(xc_ref[...], wb_ref[...],
                     preferred_element_type=jnp.float32)
        cond = jnp.dot(oh, y2, preferred_element_type=jnp.float32)
        out_ref[...] = jnp.maximum(
            jnp.dot(xo_ref[...], wt_ref[...],
                    preferred_element_type=jnp.float32)
            + cond + br_ref[...], 0.0)

    full = lambda j: (0, 0)
    return pl.pallas_call(
        body,
        grid=(nb,),
        in_specs=[
            pl.BlockSpec((BN, D), lambda j: (j, 0)),
            pl.BlockSpec((BN, 1), lambda j: (j, 0)),
            pl.BlockSpec((G, D), full),
            pl.BlockSpec((D, D), full),
            pl.BlockSpec((D, D), full),
            pl.BlockSpec((1, D), full),
        ],
        out_specs=pl.BlockSpec((BN, D), lambda j: (j, 0)),
        out_shape=jax.ShapeDtypeStruct((N, D), jnp.float32),
    )(x_orig, batch2, xc, wr_top, wr_bot, br)


def _center_update(xc3, gsum, cnt, Wc, bc, Wm, bm, G, D):
    def body(xc_ref, gs_ref, ct_ref, wc_ref, bc_ref, wm_ref, bm_ref, out_ref):
        gmp = gs_ref[...] / jnp.maximum(ct_ref[...], 1.0)
        out_ref[...] = jnp.maximum(
            jnp.dot(xc_ref[...], wc_ref[...], preferred_element_type=jnp.float32)
            + bc_ref[...]
            + jnp.dot(gmp, wm_ref[...], preferred_element_type=jnp.float32)
            + bm_ref[...], 0.0)

    return pl.pallas_call(
        body,
        out_shape=jax.ShapeDtypeStruct((G, D), jnp.float32),
    )(xc3, gsum, cnt, Wc, bc, Wm, bm)


def _overwrite(x3, cni2, center, N, D, G, BN):
    nb = N // BN

    def body(x3_ref, cni_ref, c_ref, out_ref):
        j = pl.program_id(0)
        rows = j * BN + lax.broadcasted_iota(jnp.int32, (BN, G), 0)
        m = rows == cni_ref[...]
        gi = lax.broadcasted_iota(jnp.int32, (BN, G), 1)
        gsel = jnp.max(jnp.where(m, gi, -1), axis=1)
        has = gsel >= 0
        oh2 = (gi == gsel[:, None]).astype(jnp.float32) * \
            has[:, None].astype(jnp.float32)
        repl = jnp.dot(oh2, c_ref[...], preferred_element_type=jnp.float32)
        out_ref[...] = jnp.where(has[:, None], repl, x3_ref[...])

    return pl.pallas_call(
        body,
        grid=(nb,),
        in_specs=[
            pl.BlockSpec((BN, D), lambda j: (j, 0)),
            pl.BlockSpec((1, G), lambda j: (0, 0)),
            pl.BlockSpec((G, D), lambda j: (0, 0)),
        ],
        out_specs=pl.BlockSpec((BN, D), lambda j: (j, 0)),
        out_shape=jax.ShapeDtypeStruct((N, D), jnp.float32),
    )(x3, cni2, center)


def kernel(x_orig, x, edge_index, edge_attr, center_node_index, batch,
           We, Wl, bl, Wr, br, Wc, bc, Wm, bm):
    N, D = x.shape
    E = edge_attr.shape[0]
    G = center_node_index.shape[0]
    L = We.shape[0]
    BN = 2000
    assert N % BN == 0 and E % CHUNK == 0

    src = edge_index[0]
    dst = edge_index[1]
    batch2 = batch.reshape(N, 1)
    cni2 = center_node_index.reshape(1, G)

    gather_rows = _make_gather_rows(N, G, D)
    edge_agg = _make_edge_agg(N, E, D)

    ea_t = edge_attr.T
    ee0 = _ee_matmul(ea_t, We[0])
    ee1 = _ee_matmul(ea_t, We[1])
    ee2 = _ee_matmul(ea_t, We[2])

    h = x
    parts = edge_agg(h, ee0, src, dst)
    x1 = _node_update(h, parts, Wl[0], bl[0].reshape(1, D), N, D, BN)
    xc1 = gather_rows(x1, center_node_index)
    xr = _reset(x_orig, batch2, xc1, Wr[:D], Wr[D:], br.reshape(1, D),
                N, D, G, BN)
    parts = edge_agg(xr, ee1, src, dst)
    x2 = _node_update(xr, parts, Wl[1], bl[1].reshape(1, D), N, D, BN)
    parts = edge_agg(x2, ee2, src, dst)
    x3, gsum, cnt = _node_update_pool(x2, parts, Wl[2], bl[2].reshape(1, D),
                                      batch2, N, D, G, BN)
    xc3 = gather_rows(x3, center_node_index)
    center = _center_update(xc3, gsum, cnt, Wc, bc.reshape(1, D),
                            Wm, bm.reshape(1, D), G, D)
    return _overwrite(x3, cni2, center, N, D, G, BN)


# speedup vs baseline: 2.2002x; 1.0000x over previous
import functools

import numpy as np

import jax
import jax.numpy as jnp
from jax import lax
from jax.experimental import pallas as pl
from jax.experimental.pallas import tpu as pltpu
from jax.experimental.pallas import tpu_sc as plsc

NC = 2
NS = 16
NW = NC * NS
CHUNK = 80


def _make_edge_agg(N, E, D):
    n_chunks = E // CHUNK
    per_w = n_chunks // NW
    assert per_w * NW == n_chunks
    rows_per_tile = (N // (8 * NS)) * 8
    rem = N - NS * rows_per_tile
    assert rem % 8 == 0 and rem <= CHUNK
    pieces = []
    off = 0
    while off < rows_per_tile:
        pc = min(CHUNK, rows_per_tile - off)
        assert pc % 8 == 0
        pieces.append((off, pc))
        off += pc

    mesh = plsc.VectorSubcoreMesh(core_axis_name="c", subcore_axis_name="s")

    @functools.partial(
        pl.kernel,
        out_type=jax.ShapeDtypeStruct((2 * N, D), jnp.float32),
        mesh=mesh,
        scratch_types=[
            pltpu.VMEM((CHUNK,), jnp.int32),
            pltpu.VMEM((CHUNK,), jnp.int32),
            pltpu.VMEM((CHUNK,), jnp.int32),
            pltpu.VMEM((CHUNK,), jnp.int32),
            pltpu.VMEM((CHUNK, D), jnp.float32),
            pltpu.VMEM((CHUNK, D), jnp.float32),
            pltpu.VMEM((CHUNK, D), jnp.float32),
            pltpu.VMEM((CHUNK, D), jnp.float32),
            pltpu.VMEM_SHARED((N, D), jnp.float32),
            pltpu.SemaphoreType.DMA,
            pltpu.SemaphoreType.DMA,
            pltpu.SemaphoreType.DMA,
            pltpu.SemaphoreType.DMA,
            pltpu.SemaphoreType.DMA,
            pltpu.SemaphoreType.DMA,
            pltpu.SemaphoreType.DMA,
            pltpu.SemaphoreType.DMA,
        ],
    )
    def edge_agg(h_hbm, ee_hbm, src_hbm, dst_hbm, out_hbm,
                 src0, src1, dst0, dst1, grow0, grow1, ee0, ee1, acc_sh,
                 ss0, ss1, sd0, sd1, sg0, sg1, se0, se1):
        c = lax.axis_index("c")
        s = lax.axis_index("s")
        w = c * NS + s
        srcs, dsts, grows, ees = (src0, src1), (dst0, dst1), \
            (grow0, grow1), (ee0, ee1)
        sss, sds, sgs, ses = (ss0, ss1), (sd0, sd1), (sg0, sg1), (se0, se1)

        zv = jnp.zeros((16,), jnp.float32)

        def zrow(i, carry):
            for u in range(D // 16):
                grow0[i, pl.ds(u * 16, 16)] = zv
            return carry

        lax.fori_loop(0, CHUNK, zrow, 0)
        base = s * rows_per_tile
        for po, pc in pieces:
            pltpu.sync_copy(grow0.at[pl.ds(0, pc)],
                            acc_sh.at[pl.ds(base + po, pc)])
        if rem:
            @pl.when(s == NS - 1)
            def _():
                pltpu.sync_copy(grow0.at[pl.ds(0, rem)],
                                acc_sh.at[pl.ds(NS * rows_per_tile, rem)])
        plsc.subcore_barrier()

        def ebase(r):
            return (w * per_w + r) * CHUNK

        def idx_issue(r, sl):
            pltpu.async_copy(src_hbm.at[pl.ds(ebase(r), CHUNK)],
                             srcs[sl], sss[sl])
            pltpu.async_copy(dst_hbm.at[pl.ds(ebase(r), CHUNK)],
                             dsts[sl], sds[sl])

        def idx_wait(sl):
            pltpu.make_async_copy(src_hbm.at[pl.ds(0, CHUNK)],
                                  srcs[sl], sss[sl]).wait()
            pltpu.make_async_copy(dst_hbm.at[pl.ds(0, CHUNK)],
                                  dsts[sl], sds[sl]).wait()

        def ge_issue(r, sl):
            pltpu.async_copy(ee_hbm.at[pl.ds(ebase(r), CHUNK)],
                             ees[sl], ses[sl])
            pltpu.async_copy(h_hbm.at[srcs[sl]], grows[sl], sgs[sl])

        def ge_wait(sl):
            pltpu.make_async_copy(ee_hbm.at[pl.ds(0, CHUNK)],
                                  ees[sl], ses[sl]).wait()
            pltpu.make_async_copy(h_hbm.at[srcs[sl]],
                                  grows[sl], sgs[sl]).wait()

        def step(r, cur):
            nxt = 1 - cur

            @pl.when(r < per_w - 1)
            def _():
                idx_wait(nxt)
                ge_issue(r + 1, nxt)

            ge_wait(cur)
            gv, ev = grows[cur], ees[cur]

            def crow(i, carry):
                for u in range(D // 16):
                    sl = pl.ds(u * 16, 16)
                    gv[i, sl] = jnp.maximum(gv[i, sl] + ev[i, sl], 0.0)
                return carry

            lax.fori_loop(0, CHUNK, crow, 0)
            pltpu.sync_copy(gv, acc_sh.at[dsts[cur]], add=True)

            @pl.when(r < per_w - 2)
            def _():
                idx_issue(r + 2, cur)

        idx_issue(0, 0)
        idx_wait(0)
        ge_issue(0, 0)
        idx_issue(1, 1)

        def body(r, carry):
            @pl.when(lax.rem(r, 2) == 0)
            def _():
                step(r, 0)

            @pl.when(lax.rem(r, 2) == 1)
            def _():
                step(r, 1)

            return carry

        lax.fori_loop(0, per_w, body, 0)

        plsc.subcore_barrier()
        for po, pc in pieces:
            pltpu.sync_copy(acc_sh.at[pl.ds(base + po, pc)],
                            out_hbm.at[pl.ds(c * N + base + po, pc)])
        if rem:
            @pl.when(s == NS - 1)
            def _():
                pltpu.sync_copy(acc_sh.at[pl.ds(NS * rows_per_tile, rem)],
                                out_hbm.at[pl.ds(c * N + NS * rows_per_tile, rem)])

    return edge_agg


def _make_gather_rows(N, G, D):
    mesh = plsc.VectorSubcoreMesh(core_axis_name="c", subcore_axis_name="s")

    @functools.partial(
        pl.kernel,
        out_type=jax.ShapeDtypeStruct((G, D), jnp.float32),
        mesh=mesh,
        scratch_types=[
            pltpu.VMEM((G,), jnp.int32),
            pltpu.VMEM((G, D), jnp.float32),
            pltpu.SemaphoreType.DMA,
        ],
    )
    def gather_rows(tab_hbm, idx_hbm, out_hbm, idx_v, rows_v, sem):
        c = lax.axis_index("c")
        s = lax.axis_index("s")

        @pl.when(jnp.logical_and(c == 0, s == 0))
        def _():
            pltpu.sync_copy(idx_hbm, idx_v)
            pltpu.async_copy(tab_hbm.at[idx_v], rows_v, sem).wait()
            pltpu.sync_copy(rows_v, out_hbm)

    return gather_rows


def _ee_matmul(edge_attr_t, we_k):
    DE, E = edge_attr_t.shape
    D = we_k.shape[1]
    BE = 2560
    assert E % BE == 0
    nj = E // BE

    def body(ea_ref, we_ref, out_ref):
        out_ref[...] = lax.dot_general(
            ea_ref[...], we_ref[...], (((0,), (0,)), ((), ())),
            preferred_element_type=jnp.float32)

    return pl.pallas_call(
        body,
        grid=(nj,),
        in_specs=[
            pl.BlockSpec((DE, BE), lambda j: (0, j)),
            pl.BlockSpec((DE, D), lambda j: (0, 0)),
        ],
        out_specs=pl.BlockSpec((BE, D), lambda j: (j, 0)),
        out_shape=jax.ShapeDtypeStruct((E, D), jnp.float32),
    )(edge_attr_t, we_k)


def _node_update(h, parts, wl, bl, N, D, BN, wl_p=None, bl_p=None):
    nb = N // BN
    emit_bf = wl_p is not None

    def body(h_ref, a_ref, b_ref, w_ref, b2_ref, *rest):
        acc = h_ref[...] + a_ref[...] + b_ref[...]
        if emit_bf:
            wp_ref, bp_ref, out_ref, outb_ref = rest
            outb_ref[...] = jnp.maximum(
                jnp.dot(acc, wp_ref[...], preferred_element_type=jnp.float32)
                + bp_ref[...], 0.0).astype(jnp.bfloat16)
        else:
            (out_ref,) = rest
        out_ref[...] = jnp.maximum(
            jnp.dot(acc, w_ref[...], preferred_element_type=jnp.float32)
            + b2_ref[...], 0.0)

    in_specs = [
        pl.BlockSpec((BN, D), lambda j: (j, 0)),
        pl.BlockSpec((BN, D), lambda j: (j, 0)),
        pl.BlockSpec((BN, D), lambda j: (nb + j, 0)),
        pl.BlockSpec((D, D), lambda j: (0, 0)),
        pl.BlockSpec((1, D), lambda j: (0, 0)),
    ]
    args = [h, parts, parts, wl, bl]
    out_specs = pl.BlockSpec((BN, D), lambda j: (j, 0))
    out_shape = jax.ShapeDtypeStruct((N, D), jnp.float32)
    if emit_bf:
        in_specs += [pl.BlockSpec((D, D), lambda j: (0, 0)),
                     pl.BlockSpec((1, D), lambda j: (0, 0))]
        args += [wl_p, bl_p]
        out_specs = [out_specs, pl.BlockSpec((BN, D), lambda j: (j, 0))]
        out_shape = [out_shape, jax.ShapeDtypeStruct((N, D), jnp.bfloat16)]

    return pl.pallas_call(
        body, grid=(nb,), in_specs=in_specs,
        out_specs=out_specs, out_shape=out_shape,
    )(*args)


def _node_update_pool(h, parts, wl, bl, batch2, N, D, G, BN):
    nb = N // BN

    def body(h_ref, a_ref, b_ref, w_ref, b2_ref, bt_ref,
             out_ref, gsum_ref, cnt_ref):
        acc = h_ref[...] + a_ref[...] + b_ref[...]
        x3 = jnp.maximum(
            jnp.dot(acc, w_ref[...], preferred_element_type=jnp.float32)
            + b2_ref[...], 0.0)
        out_ref[...] = x3
        gi = lax.broadcasted_iota(jnp.int32, (BN, G), 1)
        oh = (bt_ref[...] == gi).astype(jnp.float32)

        @pl.when(pl.program_id(0) == 0)
        def _():
            gsum_ref[...] = jnp.zeros((G, D), jnp.float32)
            cnt_ref[...] = jnp.zeros((G, D), jnp.float32)

        gsum_ref[...] += lax.dot_general(
            oh, x3, (((0,), (0,)), ((), ())),
            preferred_element_type=jnp.float32)
        cnt_ref[...] += jnp.broadcast_to(
            jnp.sum(oh, axis=0)[:, None], (G, D))

    return pl.pallas_call(
        body,
        grid=(nb,),
        in_specs=[
            pl.BlockSpec((BN, D), lambda j: (j, 0)),
            pl.BlockSpec((BN, D), lambda j: (j, 0)),
            pl.BlockSpec((BN, D), lambda j: (nb + j, 0)),
            pl.BlockSpec((D, D), lambda j: (0, 0)),
            pl.BlockSpec((1, D), lambda j: (0, 0)),
            pl.BlockSpec((BN, 1), lambda j: (j, 0)),
        ],
        out_specs=[
            pl.BlockSpec((BN, D), lambda j: (j, 0)),
            pl.BlockSpec((G, D), lambda j: (0, 0)),
            pl.BlockSpec((G, D), lambda j: (0, 0)),
        ],
        out_shape=[
            jax.ShapeDtypeStruct((N, D), jnp.float32),
            jax.ShapeDtypeStruct((G, D), jnp.float32),
            jax.ShapeDtypeStruct((G, D), jnp.float32),
        ],
    )(h, parts, parts, wl, bl, batch2)


def _reset(x_orig, batch2, xc, wr_top, wr_bot, br, N, D, G, BN):
    nb = N // BN

    def body(xo_ref, bt_ref, xc_ref, wt_ref, wb_ref, br_ref, out_ref):
        gi = lax.broadcasted_iota(jnp.int32, (BN, G), 1)
        oh = (bt_ref[...] == gi).astype(jnp.float32)
        y2 = jnp.dot(xc_ref[...], wb_ref[...],
                     preferred_element_type=jnp.float32)
        cond = jnp.dot(oh, y2, preferred_element_type=jnp.float32)
        out_ref[...] = jnp.maximum(
            jnp.dot(xo_ref[...], wt_ref[...],
                    preferred_element_type=jnp.float32)
            + cond + br_ref[...], 0.0)

    full = lambda j: (0, 0)
    return pl.pallas_call(
        body,
        grid=(nb,),
        in_specs=[
            pl.BlockSpec((BN, D), lambda j: (j, 0)),
            pl.BlockSpec((BN, 1), lambda j: (j, 0)),
            pl.BlockSpec((G, D), full),
            pl.BlockSpec((D, D), full),
            pl.BlockSpec((D, D), full),
            pl.BlockSpec((1, D), full),
        ],
        out_specs=pl.BlockSpec((BN, D), lambda j: (j, 0)),
        out_shape=jax.ShapeDtypeStruct((N, D), jnp.float32),
    )(x_orig, batch2, xc, wr_top, wr_bot, br)


def _center_update(xc3, gsum, cnt, Wc, bc, Wm, bm, G, D):
    def body(xc_ref, gs_ref, ct_ref, wc_ref, bc_ref, wm_ref, bm_ref, out_ref):
        gmp = gs_ref[...] / jnp.maximum(ct_ref[...], 1.0)
        out_ref[...] = jnp.maximum(
            jnp.dot(xc_ref[...], wc_ref[...], preferred_element_type=jnp.float32)
            + bc_ref[...]
            + jnp.dot(gmp, wm_ref[...], preferred_element_type=jnp.float32)
            + bm_ref[...], 0.0)

    return pl.pallas_call(
        body,
        out_shape=jax.ShapeDtypeStruct((G, D), jnp.float32),
    )(xc3, gsum, cnt, Wc, bc, Wm, bm)


def _overwrite(x3, cni2, center, N, D, G, BN):
    nb = N // BN

    def body(x3_ref, cni_ref, c_ref, out_ref):
        j = pl.program_id(0)
        rows = j * BN + lax.broadcasted_iota(jnp.int32, (BN, G), 0)
        m = rows == cni_ref[...]
        gi = lax.broadcasted_iota(jnp.int32, (BN, G), 1)
        gsel = jnp.max(jnp.where(m, gi, -1), axis=1)
        has = gsel >= 0
        oh2 = (gi == gsel[:, None]).astype(jnp.float32) * \
            has[:, None].astype(jnp.float32)
        repl = jnp.dot(oh2, c_ref[...], preferred_element_type=jnp.float32)
        out_ref[...] = jnp.where(has[:, None], repl, x3_ref[...])

    return pl.pallas_call(
        body,
        grid=(nb,),
        in_specs=[
            pl.BlockSpec((BN, D), lambda j: (j, 0)),
            pl.BlockSpec((1, G), lambda j: (0, 0)),
            pl.BlockSpec((G, D), lambda j: (0, 0)),
        ],
        out_specs=pl.BlockSpec((BN, D), lambda j: (j, 0)),
        out_shape=jax.ShapeDtypeStruct((N, D), jnp.float32),
    )(x3, cni2, center)


def kernel(x_orig, x, edge_index, edge_attr, center_node_index, batch,
           We, Wl, bl, Wr, br, Wc, bc, Wm, bm):
    N, D = x.shape
    E = edge_attr.shape[0]
    G = center_node_index.shape[0]
    L = We.shape[0]
    BN = 2000
    assert N % BN == 0 and E % CHUNK == 0

    src = edge_index[0]
    dst = edge_index[1]
    batch2 = batch.reshape(N, 1)
    cni2 = center_node_index.reshape(1, G)

    gather_rows = _make_gather_rows(N, G, D)
    edge_agg = _make_edge_agg(N, E, D)

    ea_t = edge_attr.T
    ee0 = _ee_matmul(ea_t, We[0])

    h = x
    parts = edge_agg(h, ee0, src, dst)
    ee1 = _ee_matmul(ea_t, We[1])
    ee2 = _ee_matmul(ea_t, We[2])
    x1 = _node_update(h, parts, Wl[0], bl[0].reshape(1, D), N, D, BN)
    xc1 = gather_rows(x1, center_node_index)
    xr = _reset(x_orig, batch2, xc1, Wr[:D], Wr[D:], br.reshape(1, D),
                N, D, G, BN)
    parts = edge_agg(xr, ee1, src, dst)
    x2 = _node_update(xr, parts, Wl[1], bl[1].reshape(1, D), N, D, BN)
    parts = edge_agg(x2, ee2, src, dst)
    x3, gsum, cnt = _node_update_pool(x2, parts, Wl[2], bl[2].reshape(1, D),
                                      batch2, N, D, G, BN)
    xc3 = gather_rows(x3, center_node_index)
    center = _center_update(xc3, gsum, cnt, Wc, bc.reshape(1, D),
                            Wm, bm.reshape(1, D), G, D)
    return _overwrite(x3, cni2, center, N, D, G, BN)
